# Initial kernel scaffold; baseline (speedup 1.0000x reference)
import jax
import jax.numpy as jnp
from jax.experimental import pallas as pl


def kernel(u_id, i_id, features, user_emb, user_bias, item_emb, item_bias, feat_u, feat_i, mean):
    raise NotImplementedError("write your pallas kernel here")


# speedup vs baseline: 1.5644x; 1.5644x over previous
import functools

import jax
import jax.numpy as jnp
from jax import lax
from jax.experimental import pallas as pl
from jax.experimental.pallas import tpu as pltpu
from jax.experimental.pallas import tpu_sc as plsc

L = 16


def _build(B, NF, FV, FE, NI, EMB):
    NC, NS = 2, 16
    NW = NC * NS
    PW = B // NW
    C = 16
    R = PW // C
    CH = 4
    CHN = (C * NF) // CH
    assert C * NF == CH * CHN and CHN % 8 == 0 and CHN <= 128

    mesh = plsc.VectorSubcoreMesh(
        core_axis_name="c", subcore_axis_name="s",
        num_cores=NC, num_subcores=NS)

    @functools.partial(
        pl.kernel,
        out_type=jax.ShapeDtypeStruct((B,), jnp.float32),
        mesh=mesh,
        compiler_params=pltpu.CompilerParams(
            needs_layout_passes=False, use_tc_tiling_on_sc=False),
        scratch_types=[
            pltpu.VMEM((R * CH, CHN), jnp.int32),
            pltpu.VMEM((R * CH, CHN), jnp.int32),
            pltpu.VMEM((R, C), jnp.int32),
            pltpu.VMEM((R, C), jnp.int32),
            pltpu.VMEM((C * NF, FE), jnp.float32),
            pltpu.VMEM((C * NF, FE), jnp.float32),
            pltpu.VMEM((C, EMB), jnp.float32),
            pltpu.VMEM((C, EMB), jnp.float32),
            pltpu.VMEM((C,), jnp.float32),
            pltpu.VMEM((C,), jnp.float32),
            pltpu.VMEM((L,), jnp.float32),
            pltpu.VMEM((PW,), jnp.float32),
            pltpu.SemaphoreType.DMA,
        ],
    )
    def mf_kernel(fu_tab, fi_tab, uemb, iemb, ubias, ibias,
                  uid, iid, fuidx, fiidx, mean16, out,
                  idx_fu_v, idx_fi_v, idx_u_v, idx_i_v,
                  fu_rows, fi_rows, u_rows, i_rows, bu_v, bi_v,
                  mean_v, out_v, sem):
        wid = lax.axis_index("s") * NC + lax.axis_index("c")

        pltpu.sync_copy(fuidx.at[wid], idx_fu_v)
        pltpu.sync_copy(fiidx.at[wid], idx_fi_v)
        pltpu.sync_copy(uid.at[wid], idx_u_v)
        pltpu.sync_copy(iid.at[wid], idx_i_v)
        pltpu.sync_copy(mean16, mean_v)

        def round_body(r, carry):
            cps = []
            for c in range(CH):
                cps.append(pltpu.async_copy(
                    fu_tab.at[idx_fu_v.at[r * CH + c]],
                    fu_rows.at[pl.ds(c * CHN, CHN)], sem))
                cps.append(pltpu.async_copy(
                    fi_tab.at[idx_fi_v.at[r * CH + c]],
                    fi_rows.at[pl.ds(c * CHN, CHN)], sem))
            cps.append(pltpu.async_copy(uemb.at[idx_u_v.at[r]], u_rows, sem))
            cps.append(pltpu.async_copy(iemb.at[idx_i_v.at[r]], i_rows, sem))
            cps.append(pltpu.async_copy(ubias.at[idx_u_v.at[r]], bu_v, sem))
            cps.append(pltpu.async_copy(ibias.at[idx_i_v.at[r]], bi_v, sem))
            for cp in cps:
                cp.wait()

            lanes = lax.iota(jnp.int32, L)

            def elem_body(e, res):
                base = e * NF
                acc = jnp.zeros((L,), jnp.float32)
                for j in range(NF):
                    row = base + j
                    for h in range(FE // L):
                        acc = acc + (fu_rows[row, pl.ds(h * L, L)]
                                     * fi_rows[row, pl.ds(h * L, L)])
                for h in range(EMB // L):
                    acc = acc + (u_rows[e, pl.ds(h * L, L)]
                                 * i_rows[e, pl.ds(h * L, L)])
                return res + jnp.where(lanes == e, jnp.sum(acc), 0.0)

            res0 = bu_v[:] + bi_v[:] + mean_v[:]
            res = lax.fori_loop(0, C, elem_body, res0, unroll=True)
            out_v[pl.ds(r * C, C)] = res
            return carry

        lax.fori_loop(0, R, round_body, 0)
        pltpu.sync_copy(out_v, out.at[pl.ds(wid * PW, PW)])

    return mf_kernel


def kernel(u_id, i_id, features, user_emb, user_bias, item_emb, item_bias,
           feat_u, feat_i, mean):
    B = u_id.shape[0]
    NF = features.shape[1]
    FV, FE = feat_u.shape[1], feat_u.shape[2]
    NI = feat_i.shape[1]
    EMB = user_emb.shape[1]
    NW = 32
    PW = B // NW
    C = 16
    R = PW // C
    CH = 4
    CHN = (C * NF) // CH

    f32i = jnp.int32
    fu_idx = (features.astype(f32i)
              + jnp.arange(NF, dtype=f32i)[None, :] * FV).reshape(NW, R * CH, CHN)
    fi_idx = (i_id.astype(f32i)[:, None]
              + jnp.arange(NF, dtype=f32i)[None, :] * NI).reshape(NW, R * CH, CHN)
    uid32 = u_id.astype(f32i).reshape(NW, R, C)
    iid32 = i_id.astype(f32i).reshape(NW, R, C)
    fu_tab = feat_u.reshape(NF * FV, FE)
    fi_tab = feat_i.reshape(NF * NI, FE)
    ub = user_bias.reshape(-1)
    ib = item_bias.reshape(-1)
    mean16 = jnp.broadcast_to(mean.astype(jnp.float32), (L,))

    fn = _build(B, NF, FV, FE, NI, EMB)
    return fn(fu_tab, fi_tab, user_emb, item_emb, ub, ib,
              uid32, iid32, fu_idx, fi_idx, mean16)
